# Initial kernel scaffold; baseline (speedup 1.0000x reference)
import jax
import jax.numpy as jnp
from jax.experimental import pallas as pl


def kernel(h, t_ei, t_ea, d_ei, d_ea, Wt1, bt1, Wt2, bt2, Wd1, bd1, Wd2, bd2, Wu, bu, gamma, beta):
    raise NotImplementedError("write your pallas kernel here")


# speedup vs baseline: 2.0794x; 2.0794x over previous
import jax
import jax.numpy as jnp
from jax import lax
from jax.experimental import pallas as pl
from jax.experimental.pallas import tpu as pltpu
from jax.experimental.pallas import tpu_sc as plsc

NC, NS = 2, 16
NW = NC * NS
CH = 80
GRP = 5
SGRP = 2
TRASH = 8


def _gather_body(h_ref, src_ref, out_ref, idx_v, rows_v, gsem, ssem):
    c = lax.axis_index("c")
    s = lax.axis_index("s")
    wid = s * NC + c
    e = src_ref.shape[0] // 2
    per_tile = e // NW
    n_groups = per_tile // (CH * GRP)
    for p in range(2):
        base = p * e + wid * per_tile
        pltpu.sync_copy(src_ref.at[pl.ds(base, per_tile)], idx_v)

        @pl.loop(0, n_groups)
        def _grp(g):
            row0 = g * (CH * GRP)
            gets = []
            for i in range(GRP):
                gets.append(pltpu.async_copy(
                    h_ref.at[idx_v.at[pl.ds(row0 + i * CH, CH)]],
                    rows_v.at[i], gsem))
            for cp in gets:
                cp.wait()
            puts = []
            for i in range(GRP):
                puts.append(pltpu.async_copy(
                    rows_v.at[i],
                    out_ref.at[p, pl.ds(wid * per_tile + row0 + i * CH, CH), :],
                    ssem))
            for cp in puts:
                cp.wait()


def _scatter_body(msg_ref, dsti_ref, agg_ref, cnt_ref,
                  idx_v, rows_v, buf_v,
                  acc_sh, isem, gsem, asem):
    c = lax.axis_index("c")
    s = lax.axis_index("s")
    e = msg_ref.shape[1]
    n = agg_ref.shape[1]
    d = agg_ref.shape[2]
    half = n // 2
    per_tile = e // NS
    rows_per_tile = half // NS
    n_sub = rows_per_tile // 5

    @pl.loop(0, n_sub)
    def _z(r):
        for j in range(d // 16):
            buf_v[r, pl.ds(j * 16, 16)] = jnp.zeros((16,), jnp.float32)

    base = s * per_tile
    n_groups = per_tile // (CH * SGRP)

    for phase in range(2):
        out_ref = agg_ref if phase == 0 else cnt_ref
        if phase == 1:
            @pl.loop(0, CH)
            def _ofill(r):
                for i in range(SGRP):
                    for j in range(d // 16):
                        rows_v[i, r, pl.ds(j * 16, 16)] = jnp.ones((16,), jnp.float32)

        for hf in range(2):
            lo = hf * half
            for j in range(5):
                pltpu.sync_copy(buf_v, acc_sh.at[pl.ds(s * rows_per_tile + j * n_sub, n_sub), :])

            @pl.when(s == 0)
            def _zt():
                pltpu.sync_copy(buf_v.at[pl.ds(0, TRASH), :], acc_sh.at[pl.ds(half, TRASH), :])

            plsc.subcore_barrier()

            @pl.loop(0, n_groups)
            def _grp(g):
                row0 = base + g * (CH * SGRP)
                gets = []
                for i in range(SGRP):
                    gets.append(pltpu.async_copy(
                        dsti_ref.at[pl.ds((2 * hf + c) * e + row0 + i * CH, CH)],
                        idx_v.at[i], isem))
                    if phase == 0:
                        gets.append(pltpu.async_copy(
                            msg_ref.at[c, pl.ds(row0 + i * CH, CH), :],
                            rows_v.at[i], gsem))
                for cp in gets:
                    cp.wait()
                adds = []
                for i in range(SGRP):
                    adds.append(pltpu.async_copy(
                        rows_v.at[i], acc_sh.at[idx_v.at[i]], asem, add=True))
                for cp in adds:
                    cp.wait()

            plsc.subcore_barrier()

            for j in range(5):
                r0 = s * rows_per_tile + j * n_sub
                pltpu.sync_copy(acc_sh.at[pl.ds(r0, n_sub), :], buf_v)
                pltpu.sync_copy(buf_v, out_ref.at[c, pl.ds(lo + r0, n_sub), :])

            @pl.loop(0, n_sub)
            def _z2(r):
                for j in range(d // 16):
                    buf_v[r, pl.ds(j * 16, 16)] = jnp.zeros((16,), jnp.float32)

            plsc.subcore_barrier()


def _mlp_body(hs_ref, ea_ref, w1h_ref, w1e_ref, b1_ref, w2_ref, b2_ref, out_ref):
    x = jnp.dot(hs_ref[0], w1h_ref[0], preferred_element_type=jnp.float32)
    x = x + lax.dot_general(ea_ref[0], w1e_ref[0],
                            (((0,), (0,)), ((), ())),
                            preferred_element_type=jnp.float32)
    x = jnp.maximum(x + b1_ref[0], 0.0)
    out_ref[0] = jnp.dot(x, w2_ref[0], preferred_element_type=jnp.float32) + b2_ref[0]


def _final_body(h_ref, at_ref, ad_ref, ct_ref, cd_ref, wut_ref, wud_ref, bu_ref,
                g_ref, b_ref, out_ref):
    at = at_ref[...] / jnp.maximum(ct_ref[:, 0:1], 1.0)
    ad = ad_ref[...] / jnp.maximum(cd_ref[:, 0:1], 1.0)
    upd = jnp.dot(at, wut_ref[...], preferred_element_type=jnp.float32)
    upd = upd + jnp.dot(ad, wud_ref[...], preferred_element_type=jnp.float32)
    x = h_ref[...] + upd + bu_ref[0]
    mu = jnp.mean(x, axis=1, keepdims=True)
    xc = x - mu
    var = jnp.mean(xc * xc, axis=1, keepdims=True)
    y = xc * lax.rsqrt(var + 1e-5)
    out_ref[...] = y * g_ref[0] + b_ref[0]


def _sc_mesh():
    return plsc.VectorSubcoreMesh(core_axis_name="c", subcore_axis_name="s",
                                  num_cores=NC, num_subcores=NS)


def _sc_gather(h2, src):
    n, d = h2.shape
    e = src.shape[0] // 2
    gather = pl.kernel(
        _gather_body,
        out_type=jax.ShapeDtypeStruct((2, e, d), jnp.float32),
        mesh=_sc_mesh(),
        scratch_types=[
            pltpu.VMEM((e // NW,), jnp.int32),
            pltpu.VMEM((GRP, CH, d), jnp.float32),
            pltpu.SemaphoreType.DMA,
            pltpu.SemaphoreType.DMA,
        ],
    )
    return gather(h2, src)


def _sc_scatter(msg, dsti, n):
    _, e, d = msg.shape
    npad = -(-n // (NS * 40 * 2)) * (NS * 40 * 2)
    half = npad // 2
    scatter = pl.kernel(
        _scatter_body,
        out_type=(jax.ShapeDtypeStruct((2, npad, d), jnp.float32),
                  jax.ShapeDtypeStruct((2, npad, d), jnp.float32)),
        mesh=_sc_mesh(),
        scratch_types=[
            pltpu.VMEM((SGRP, CH), jnp.int32),
            pltpu.VMEM((SGRP, CH, d), jnp.float32),
            pltpu.VMEM((half // NS // 5, d), jnp.float32),
            pltpu.VMEM_SHARED((half + TRASH, d), jnp.float32),
            pltpu.SemaphoreType.DMA,
            pltpu.SemaphoreType.DMA,
            pltpu.SemaphoreType.DMA,
        ],
    )
    return scatter(msg, dsti)


def kernel(h, t_ei, t_ea, d_ei, d_ea, Wt1, bt1, Wt2, bt2, Wd1, bd1, Wd2, bd2,
           Wu, bu, gamma, beta):
    _, n, d = h.shape
    e = t_ei.shape[1]
    ef = t_ea.shape[1]
    h2 = h[0]

    src = jnp.concatenate([t_ei[0], d_ei[0]]).astype(jnp.int32)
    dst = jnp.concatenate([t_ei[1], d_ei[1]]).astype(jnp.int32)
    ea_t = jnp.stack([t_ea.T, d_ea.T])
    w1h = jnp.stack([Wt1[:d], Wd1[:d]])
    w1e = jnp.stack([Wt1[d:], Wd1[d:]])
    b1 = jnp.stack([bt1, bd1]).reshape(2, 1, 2 * d)
    w2 = jnp.stack([Wt2, Wd2])
    b2 = jnp.stack([bt2, bd2]).reshape(2, 1, d)

    hs = _sc_gather(h2, src)

    bs = 2560
    nb = e // bs
    msg = pl.pallas_call(
        _mlp_body,
        grid=(2, nb),
        in_specs=[
            pl.BlockSpec((1, bs, d), lambda p, i: (p, i, 0)),
            pl.BlockSpec((1, ef, bs), lambda p, i: (p, 0, i)),
            pl.BlockSpec((1, d, 2 * d), lambda p, i: (p, 0, 0)),
            pl.BlockSpec((1, ef, 2 * d), lambda p, i: (p, 0, 0)),
            pl.BlockSpec((1, 1, 2 * d), lambda p, i: (p, 0, 0)),
            pl.BlockSpec((1, 2 * d, d), lambda p, i: (p, 0, 0)),
            pl.BlockSpec((1, 1, d), lambda p, i: (p, 0, 0)),
        ],
        out_specs=pl.BlockSpec((1, bs, d), lambda p, i: (p, i, 0)),
        out_shape=jax.ShapeDtypeStruct((2, e, d), jnp.float32),
    )(hs, ea_t, w1h, w1e, b1, w2, b2)

    npad = -(-n // (NS * 40 * 2)) * (NS * 40 * 2)
    half = npad // 2
    rel0 = jnp.where(dst < half, dst, half)
    rel1 = jnp.where(dst >= half, dst - half, half)
    dsti = jnp.concatenate([rel0, rel1])
    aggsum, cnt = _sc_scatter(msg, dsti, n)

    bn = n // 10
    out = pl.pallas_call(
        _final_body,
        grid=(n // bn,),
        in_specs=[
            pl.BlockSpec((bn, d), lambda i: (i, 0)),
            pl.BlockSpec((bn, d), lambda i: (i, 0)),
            pl.BlockSpec((bn, d), lambda i: (i, 0)),
            pl.BlockSpec((bn, d), lambda i: (i, 0)),
            pl.BlockSpec((bn, d), lambda i: (i, 0)),
            pl.BlockSpec((d, d), lambda i: (0, 0)),
            pl.BlockSpec((d, d), lambda i: (0, 0)),
            pl.BlockSpec((1, d), lambda i: (0, 0)),
            pl.BlockSpec((1, d), lambda i: (0, 0)),
            pl.BlockSpec((1, d), lambda i: (0, 0)),
        ],
        out_specs=pl.BlockSpec((bn, d), lambda i: (i, 0)),
        out_shape=jax.ShapeDtypeStruct((n, d), jnp.float32),
    )(h2, aggsum[0], aggsum[1], cnt[0], cnt[1], Wu[:d], Wu[d:], bu.reshape(1, d),
      gamma.reshape(1, d), beta.reshape(1, d))

    return out[None]


# speedup vs baseline: 2.0884x; 1.0043x over previous
import jax
import jax.numpy as jnp
from jax import lax
from jax.experimental import pallas as pl
from jax.experimental.pallas import tpu as pltpu
from jax.experimental.pallas import tpu_sc as plsc

NC, NS = 2, 16
NW = NC * NS
CH = 80
GRP = 5
SGRP = 2
TRASH = 8


def _gather_body(h_ref, src_ref, out_ref, idx_v, rows_v, gsem, ssem):
    c = lax.axis_index("c")
    s = lax.axis_index("s")
    wid = s * NC + c
    e = src_ref.shape[0] // 2
    per_tile = e // NW
    n_groups = per_tile // (CH * GRP)
    for p in range(2):
        base = p * e + wid * per_tile
        pltpu.sync_copy(src_ref.at[pl.ds(base, per_tile)], idx_v)

        @pl.loop(0, n_groups)
        def _grp(g):
            row0 = g * (CH * GRP)
            gets = []
            for i in range(GRP):
                gets.append(pltpu.async_copy(
                    h_ref.at[idx_v.at[pl.ds(row0 + i * CH, CH)]],
                    rows_v.at[i], gsem))
            for cp in gets:
                cp.wait()
            puts = []
            for i in range(GRP):
                puts.append(pltpu.async_copy(
                    rows_v.at[i],
                    out_ref.at[p, pl.ds(wid * per_tile + row0 + i * CH, CH), :],
                    ssem))
            for cp in puts:
                cp.wait()


def _scatter_body(msg_ref, dsti_ref, agg_ref, cnt_ref,
                  idx_v, rows_v, buf_v,
                  acc_sh, isem, gsem, asem):
    c = lax.axis_index("c")
    s = lax.axis_index("s")
    e = msg_ref.shape[1]
    n = agg_ref.shape[1]
    d = agg_ref.shape[2]
    half = n // 2
    per_tile = e // NS
    rows_per_tile = half // NS
    n_sub = rows_per_tile // 5

    @pl.loop(0, n_sub)
    def _z(r):
        for j in range(d // 16):
            buf_v[r, pl.ds(j * 16, 16)] = jnp.zeros((16,), jnp.float32)

    base = s * per_tile
    n_groups = per_tile // (CH * SGRP)

    for phase in range(2):
        out_ref = agg_ref if phase == 0 else cnt_ref
        if phase == 1:
            @pl.loop(0, CH)
            def _ofill(r):
                for i in range(SGRP):
                    for j in range(d // 16):
                        rows_v[i, r, pl.ds(j * 16, 16)] = jnp.ones((16,), jnp.float32)

        for hf in range(2):
            lo = hf * half
            for j in range(5):
                pltpu.sync_copy(buf_v, acc_sh.at[pl.ds(s * rows_per_tile + j * n_sub, n_sub), :])

            @pl.when(s == 0)
            def _zt():
                pltpu.sync_copy(buf_v.at[pl.ds(0, TRASH), :], acc_sh.at[pl.ds(half, TRASH), :])

            plsc.subcore_barrier()

            @pl.loop(0, n_groups)
            def _grp(g):
                row0 = base + g * (CH * SGRP)
                gets = []
                for i in range(SGRP):
                    gets.append(pltpu.async_copy(
                        dsti_ref.at[pl.ds((2 * hf + c) * e + row0 + i * CH, CH)],
                        idx_v.at[i], isem))
                    if phase == 0:
                        gets.append(pltpu.async_copy(
                            msg_ref.at[c, pl.ds(row0 + i * CH, CH), :],
                            rows_v.at[i], gsem))
                for cp in gets:
                    cp.wait()
                adds = []
                for i in range(SGRP):
                    adds.append(pltpu.async_copy(
                        rows_v.at[i], acc_sh.at[idx_v.at[i]], asem, add=True))
                for cp in adds:
                    cp.wait()

            plsc.subcore_barrier()

            for j in range(5):
                r0 = s * rows_per_tile + j * n_sub
                pltpu.sync_copy(acc_sh.at[pl.ds(r0, n_sub), :], buf_v)
                pltpu.sync_copy(buf_v, out_ref.at[c, pl.ds(lo + r0, n_sub), :])

            @pl.loop(0, n_sub)
            def _z2(r):
                for j in range(d // 16):
                    buf_v[r, pl.ds(j * 16, 16)] = jnp.zeros((16,), jnp.float32)

            plsc.subcore_barrier()


def _mlp_body(hs_ref, ea_ref, w1h_ref, w1e_ref, b1_ref, w2_ref, b2_ref, out_ref):
    x = jnp.dot(hs_ref[0].astype(jnp.bfloat16), w1h_ref[0],
                preferred_element_type=jnp.float32)
    x = x + lax.dot_general(ea_ref[0], w1e_ref[0],
                            (((0,), (0,)), ((), ())),
                            preferred_element_type=jnp.float32)
    x = jnp.maximum(x + b1_ref[0], 0.0).astype(jnp.bfloat16)
    out_ref[0] = jnp.dot(x, w2_ref[0], preferred_element_type=jnp.float32) + b2_ref[0]


def _final_body(h_ref, at_ref, ad_ref, ct_ref, cd_ref, wut_ref, wud_ref, bu_ref,
                g_ref, b_ref, out_ref):
    at = at_ref[...] / jnp.maximum(ct_ref[:, 0:1], 1.0)
    ad = ad_ref[...] / jnp.maximum(cd_ref[:, 0:1], 1.0)
    upd = jnp.dot(at, wut_ref[...], preferred_element_type=jnp.float32)
    upd = upd + jnp.dot(ad, wud_ref[...], preferred_element_type=jnp.float32)
    x = h_ref[...] + upd + bu_ref[0]
    mu = jnp.mean(x, axis=1, keepdims=True)
    xc = x - mu
    var = jnp.mean(xc * xc, axis=1, keepdims=True)
    y = xc * lax.rsqrt(var + 1e-5)
    out_ref[...] = y * g_ref[0] + b_ref[0]


def _sc_mesh():
    return plsc.VectorSubcoreMesh(core_axis_name="c", subcore_axis_name="s",
                                  num_cores=NC, num_subcores=NS)


def _sc_gather(h2, src):
    n, d = h2.shape
    e = src.shape[0] // 2
    gather = pl.kernel(
        _gather_body,
        out_type=jax.ShapeDtypeStruct((2, e, d), jnp.float32),
        mesh=_sc_mesh(),
        scratch_types=[
            pltpu.VMEM((e // NW,), jnp.int32),
            pltpu.VMEM((GRP, CH, d), jnp.float32),
            pltpu.SemaphoreType.DMA,
            pltpu.SemaphoreType.DMA,
        ],
    )
    return gather(h2, src)


def _sc_scatter(msg, dsti, n):
    _, e, d = msg.shape
    npad = -(-n // (NS * 40 * 2)) * (NS * 40 * 2)
    half = npad // 2
    scatter = pl.kernel(
        _scatter_body,
        out_type=(jax.ShapeDtypeStruct((2, npad, d), jnp.float32),
                  jax.ShapeDtypeStruct((2, npad, d), jnp.float32)),
        mesh=_sc_mesh(),
        scratch_types=[
            pltpu.VMEM((SGRP, CH), jnp.int32),
            pltpu.VMEM((SGRP, CH, d), jnp.float32),
            pltpu.VMEM((half // NS // 5, d), jnp.float32),
            pltpu.VMEM_SHARED((half + TRASH, d), jnp.float32),
            pltpu.SemaphoreType.DMA,
            pltpu.SemaphoreType.DMA,
            pltpu.SemaphoreType.DMA,
        ],
    )
    return scatter(msg, dsti)


def kernel(h, t_ei, t_ea, d_ei, d_ea, Wt1, bt1, Wt2, bt2, Wd1, bd1, Wd2, bd2,
           Wu, bu, gamma, beta):
    _, n, d = h.shape
    e = t_ei.shape[1]
    ef = t_ea.shape[1]
    h2 = h[0]

    src = jnp.concatenate([t_ei[0], d_ei[0]]).astype(jnp.int32)
    dst = jnp.concatenate([t_ei[1], d_ei[1]]).astype(jnp.int32)
    ea_t = jnp.stack([t_ea.T, d_ea.T]).astype(jnp.bfloat16)
    w1h = jnp.stack([Wt1[:d], Wd1[:d]]).astype(jnp.bfloat16)
    w1e = jnp.stack([Wt1[d:], Wd1[d:]]).astype(jnp.bfloat16)
    b1 = jnp.stack([bt1, bd1]).reshape(2, 1, 2 * d)
    w2 = jnp.stack([Wt2, Wd2]).astype(jnp.bfloat16)
    b2 = jnp.stack([bt2, bd2]).reshape(2, 1, d)

    hs = _sc_gather(h2, src)

    bs = 2560
    nb = e // bs
    msg = pl.pallas_call(
        _mlp_body,
        grid=(2, nb),
        in_specs=[
            pl.BlockSpec((1, bs, d), lambda p, i: (p, i, 0)),
            pl.BlockSpec((1, ef, bs), lambda p, i: (p, 0, i)),
            pl.BlockSpec((1, d, 2 * d), lambda p, i: (p, 0, 0)),
            pl.BlockSpec((1, ef, 2 * d), lambda p, i: (p, 0, 0)),
            pl.BlockSpec((1, 1, 2 * d), lambda p, i: (p, 0, 0)),
            pl.BlockSpec((1, 2 * d, d), lambda p, i: (p, 0, 0)),
            pl.BlockSpec((1, 1, d), lambda p, i: (p, 0, 0)),
        ],
        out_specs=pl.BlockSpec((1, bs, d), lambda p, i: (p, i, 0)),
        out_shape=jax.ShapeDtypeStruct((2, e, d), jnp.float32),
    )(hs, ea_t, w1h, w1e, b1, w2, b2)

    npad = -(-n // (NS * 40 * 2)) * (NS * 40 * 2)
    half = npad // 2
    rel0 = jnp.where(dst < half, dst, half)
    rel1 = jnp.where(dst >= half, dst - half, half)
    dsti = jnp.concatenate([rel0, rel1])
    aggsum, cnt = _sc_scatter(msg, dsti, n)

    bn = n // 10
    out = pl.pallas_call(
        _final_body,
        grid=(n // bn,),
        in_specs=[
            pl.BlockSpec((bn, d), lambda i: (i, 0)),
            pl.BlockSpec((bn, d), lambda i: (i, 0)),
            pl.BlockSpec((bn, d), lambda i: (i, 0)),
            pl.BlockSpec((bn, d), lambda i: (i, 0)),
            pl.BlockSpec((bn, d), lambda i: (i, 0)),
            pl.BlockSpec((d, d), lambda i: (0, 0)),
            pl.BlockSpec((d, d), lambda i: (0, 0)),
            pl.BlockSpec((1, d), lambda i: (0, 0)),
            pl.BlockSpec((1, d), lambda i: (0, 0)),
            pl.BlockSpec((1, d), lambda i: (0, 0)),
        ],
        out_specs=pl.BlockSpec((bn, d), lambda i: (i, 0)),
        out_shape=jax.ShapeDtypeStruct((n, d), jnp.float32),
    )(h2, aggsum[0], aggsum[1], cnt[0], cnt[1], Wu[:d], Wu[d:], bu.reshape(1, d),
      gamma.reshape(1, d), beta.reshape(1, d))

    return out[None]
